# Initial kernel scaffold; baseline (speedup 1.0000x reference)
import jax
import jax.numpy as jnp
from jax.experimental import pallas as pl


def kernel(loc_data, conf_data, priors, targets):
    raise NotImplementedError("write your pallas kernel here")


# speedup vs baseline: 19.1193x; 19.1193x over previous
import numpy as np
import jax
import jax.numpy as jnp
from jax.experimental import pallas as pl

_NUM_CLASSES = 21
_THRESHOLD = 0.5
_NEGPOS_RATIO = 3
_V0 = 0.1
_V1 = 0.2
_B, _P, _C, _O = 32, 8732, 21, 16
_SIGN = -(2 ** 31)


def _signed_key(i32):
    return i32 ^ ((i32 >> 31) & 0x7FFFFFFF)


def _mbl_body(tgt_ref, prior_ref, loc_ref, conf_ref,
              out_l, out_r, out_c, out_n):
    b = pl.program_id(0)

    @pl.when(b == 0)
    def _init():
        z = jnp.zeros((1, 1), jnp.float32)
        out_l[...] = z
        out_r[...] = z
        out_c[...] = z
        out_n[...] = z

    f32 = jnp.float32
    pcx = prior_ref[0:1, :]
    pcy = prior_ref[1:2, :]
    pw = prior_ref[2:3, :]
    ph = prior_ref[3:4, :]
    px1 = pcx - pw * 0.5
    py1 = pcy - ph * 0.5
    px2 = pcx + pw * 0.5
    py2 = pcy + ph * 0.5
    parea = pw * ph

    tgt = tgt_ref[0]
    tx1 = tgt[:, 0:1]
    ty1 = tgt[:, 1:2]
    tx2 = tgt[:, 2:3]
    ty2 = tgt[:, 3:4]
    tlab = tgt[:, 4:5]
    tarea = (tx2 - tx1) * (ty2 - ty1)

    ix = jnp.clip(jnp.minimum(tx2, px2) - jnp.maximum(tx1, px1), 0.0, None)
    iy = jnp.clip(jnp.minimum(ty2, py2) - jnp.maximum(ty1, py1), 0.0, None)
    inter = ix * iy
    ov = inter / (tarea + parea - inter)

    ti = jax.lax.broadcasted_iota(jnp.int32, (_O, 1), 0)
    pi = jax.lax.broadcasted_iota(jnp.int32, (1, _P), 1)

    bt_val = jnp.max(ov, axis=0, keepdims=True)
    bt_idx = jnp.min(jnp.where(ov == bt_val, ti, _O), axis=0, keepdims=True)

    ov_clone = jnp.where(ti == bt_idx, -1.0, ov)
    sb_val = jnp.max(ov_clone, axis=0, keepdims=True)
    sb_idx = jnp.min(jnp.where(ov_clone == sb_val, ti, _O),
                     axis=0, keepdims=True)

    bp_val = jnp.max(ov, axis=1, keepdims=True)
    bp_idx = jnp.min(jnp.where(ov == bp_val, pi, _P), axis=1, keepdims=True)

    hit = bp_idx == pi
    t_match = jnp.max(jnp.where(hit, ti, -1), axis=0, keepdims=True)
    forced = t_match >= 0
    ovl = jnp.where(forced, 2.0, bt_val)
    idxf = jnp.where(forced, t_match, bt_idx)

    onehot = ti == idxf
    zero = jnp.zeros((), f32)
    mx1 = jnp.sum(jnp.where(onehot, tx1, zero), axis=0, keepdims=True)
    my1 = jnp.sum(jnp.where(onehot, ty1, zero), axis=0, keepdims=True)
    mx2 = jnp.sum(jnp.where(onehot, tx2, zero), axis=0, keepdims=True)
    my2 = jnp.sum(jnp.where(onehot, ty2, zero), axis=0, keepdims=True)
    lbl = jnp.sum(jnp.where(onehot, tlab, zero), axis=0, keepdims=True)
    onehot2 = ti == sb_idx
    gx1 = jnp.sum(jnp.where(onehot2, tx1, zero), axis=0, keepdims=True)
    gy1 = jnp.sum(jnp.where(onehot2, ty1, zero), axis=0, keepdims=True)
    gx2 = jnp.sum(jnp.where(onehot2, tx2, zero), axis=0, keepdims=True)
    gy2 = jnp.sum(jnp.where(onehot2, ty2, zero), axis=0, keepdims=True)

    pos = ovl >= _THRESHOLD
    posf = pos.astype(f32)
    npos_i = jnp.sum(pos.astype(jnp.int32))
    out_n[...] = out_n[...] + npos_i.astype(f32).reshape(1, 1)

    ecx = ((mx1 + mx2) * 0.5 - pcx) / (_V0 * pw)
    ecy = ((my1 + my2) * 0.5 - pcy) / (_V0 * ph)
    ew = jnp.log((mx2 - mx1) / pw) / _V1
    eh = jnp.log((my2 - my1) / ph) / _V1
    ld0 = loc_ref[0, 0:1, :]
    ld1 = loc_ref[0, 1:2, :]
    ld2 = loc_ref[0, 2:3, :]
    ld3 = loc_ref[0, 3:4, :]

    sl1 = jnp.zeros_like(pcx)
    for ld, e in ((ld0, ecx), (ld1, ecy), (ld2, ew), (ld3, eh)):
        d = ld - e
        ad = jnp.abs(d)
        sl1 = sl1 + jnp.where(ad < 1.0, 0.5 * d * d, ad - 0.5)
    loss_l_img = jnp.sum(sl1 * posf)

    dcx = pcx + ld0 * (_V0 * pw)
    dcy = pcy + ld1 * (_V0 * ph)
    dw = pw * jnp.exp(ld2 * _V1)
    dh = ph * jnp.exp(ld3 * _V1)
    dx1 = dcx - dw * 0.5
    dx2 = dx1 + dw
    dy1 = dcy - dh * 0.5
    dy2 = dy1 + dh
    iw = jnp.clip(jnp.minimum(gx2, dx2) - jnp.maximum(gx1, dx1), 0.0, None)
    ih = jnp.clip(jnp.minimum(gy2, dy2) - jnp.maximum(gy1, dy1), 0.0, None)
    garea = (gx2 - gx1) * (gy2 - gy1)
    iog = (iw * ih) / garea
    rep = -jnp.log((1.0 - iog) + jnp.float32(1e-10))
    loss_r_img = jnp.sum(rep * posf)

    cblk = conf_ref[0]
    cmax = jnp.max(cblk, axis=0, keepdims=True)
    ssum = jnp.sum(jnp.exp(cblk - cmax), axis=0, keepdims=True)
    lse = jnp.log(ssum) + cmax
    conf_t = jnp.where(pos, lbl.astype(jnp.int32) + 1, 0)
    csel = jnp.zeros_like(pcx)
    for c in range(_C):
        csel = jnp.where(conf_t == c, cblk[c:c + 1, :], csel)
    ce = lse - csel
    ce_pos = jnp.sum(jnp.where(pos, ce, zero))

    v = jnp.where(pos, zero, ce)
    keys = _signed_key(jax.lax.bitcast_convert_type(v, jnp.int32))
    num_neg = jnp.minimum(_NEGPOS_RATIO * npos_i, _P - 1)

    x_bits = jnp.int32(0)
    for bit in range(31, -1, -1):
        m = 1 << bit
        if m >= 2 ** 31:
            m -= 2 ** 32
        trial = x_bits | jnp.int32(m)
        trial_s = trial ^ jnp.int32(_SIGN)
        cnt = jnp.sum((keys >= trial_s).astype(jnp.int32))
        x_bits = jnp.where(cnt >= num_neg, trial, x_bits)
    t_s = x_bits ^ jnp.int32(_SIGN)

    gt = keys > t_s
    g_cnt = jnp.sum(gt.astype(jnp.int32))
    sum_gt = jnp.sum(jnp.where(gt & jnp.logical_not(pos), ce, zero))
    t_f = jax.lax.bitcast_convert_type(_signed_key(t_s), f32)
    quota = (num_neg - g_cnt).astype(f32)
    tie_part = jnp.where(num_neg > g_cnt, quota * t_f, zero)
    loss_c_img = ce_pos + sum_gt + tie_part

    out_l[...] = out_l[...] + loss_l_img.reshape(1, 1)
    out_r[...] = out_r[...] + loss_r_img.reshape(1, 1)
    out_c[...] = out_c[...] + loss_c_img.reshape(1, 1)


@jax.jit
def kernel(loc_data, conf_data, priors, targets):
    loc_t = jnp.transpose(loc_data, (0, 2, 1))
    conf_t = jnp.transpose(conf_data, (0, 2, 1))
    priors_t = priors.T

    out_shapes = [jax.ShapeDtypeStruct((1, 1), jnp.float32)] * 4
    outs = pl.pallas_call(
        _mbl_body,
        grid=(_B,),
        in_specs=[
            pl.BlockSpec((1, _O, 5), lambda b: (b, 0, 0)),
            pl.BlockSpec((4, _P), lambda b: (0, 0)),
            pl.BlockSpec((1, 4, _P), lambda b: (b, 0, 0)),
            pl.BlockSpec((1, _C, _P), lambda b: (b, 0, 0)),
        ],
        out_specs=[pl.BlockSpec((1, 1), lambda b: (0, 0))] * 4,
        out_shape=out_shapes,
    )(targets, priors_t, loc_t, conf_t)
    ll, lr, lc, n = outs
    n = n[0, 0]
    return (ll[0, 0] / n, lr[0, 0] / n, lc[0, 0] / n)
